# Initial kernel scaffold; baseline (speedup 1.0000x reference)
import jax
import jax.numpy as jnp
from jax.experimental import pallas as pl


def kernel(x, table):
    raise NotImplementedError("write your pallas kernel here")


# speedup vs baseline: 11.7620x; 11.7620x over previous
import functools

import jax
import jax.numpy as jnp
from jax import lax
from jax.experimental import pallas as pl
from jax.experimental.pallas import tpu as pltpu
from jax.experimental.pallas import tpu_sc as plsc

_info = plsc.get_sparse_core_info()
_NC, _NS, _L = _info.num_cores, _info.num_subcores, _info.num_lanes
_NW = _NC * _NS

_BT = 128
_E = 16
_HW = 128 * 128
_VPAD = 1008
_CH = 2048
_BT_PER_W = _BT // _NW
_NCH = _HW // _CH
_CHUNKS = _BT_PER_W * _NCH


def _sc_embed(x_flat, tab_flat):
    mesh = plsc.VectorSubcoreMesh(core_axis_name="c", subcore_axis_name="s")

    @functools.partial(
        pl.kernel,
        out_type=jax.ShapeDtypeStruct((_BT, _E, _HW), jnp.float32),
        mesh=mesh,
        compiler_params=pltpu.CompilerParams(needs_layout_passes=False),
        scratch_types=[
            pltpu.VMEM((_VPAD * _E,), jnp.float32),
            pltpu.VMEM((2, _CH), jnp.int32),
            pltpu.VMEM((2, _E, _CH), jnp.float32),
            pltpu.SemaphoreType.DMA,
            pltpu.SemaphoreType.DMA,
            pltpu.SemaphoreType.DMA,
            pltpu.SemaphoreType.DMA,
        ],
    )
    def body(x_hbm, tab_hbm, out_hbm, tab_v, idx_v, out_v,
             isem0, isem1, osem0, osem1):
        wid = lax.axis_index("s") * _NC + lax.axis_index("c")
        isems = (isem0, isem1)
        osems = (osem0, osem1)

        pltpu.sync_copy(tab_hbm, tab_v)

        def idx_off(t):
            bt = wid * _BT_PER_W + (t // _NCH)
            return bt * _HW + (t % _NCH) * _CH

        def start_idx(t, b):
            return pltpu.async_copy(
                x_hbm.at[pl.ds(idx_off(t), _CH)], idx_v.at[b], isems[b])

        def wait_idx(b):
            pltpu.make_async_copy(
                x_hbm.at[pl.ds(0, _CH)], idx_v.at[b], isems[b]).wait()

        def start_out(t, b):
            bt = wid * _BT_PER_W + (t // _NCH)
            dst = out_hbm.at[bt, :, pl.ds((t % _NCH) * _CH, _CH)]
            return pltpu.async_copy(out_v.at[b], dst, osems[b])

        def drain_out(b):
            pltpu.make_async_copy(
                out_v.at[b], out_hbm.at[0, :, pl.ds(0, _CH)], osems[b]).wait()

        def compute(b):
            def jbody(j, carry):
                pos = j * _L
                idx = idx_v[b, pl.ds(pos, _L)]
                base = idx * _E
                for e in range(_E):
                    v = plsc.load_gather(tab_v, [base + e])
                    out_v[b, e, pl.ds(pos, _L)] = v
                return carry
            lax.fori_loop(0, _CH // _L, jbody, 0)

        start_idx(0, 0)
        start_idx(1, 1)

        def outer(tt, carry):
            for b in (0, 1):
                t = tt * 2 + b
                wait_idx(b)

                @pl.when(tt > 0)
                def _():
                    drain_out(b)

                compute(b)
                start_out(t, b)

                @pl.when(t + 2 < _CHUNKS)
                def _():
                    start_idx(t + 2, b)
            return carry

        lax.fori_loop(0, _CHUNKS // 2, outer, 0)
        drain_out(0)
        drain_out(1)

    return body(x_flat, tab_flat)


def kernel(x, table):
    x_flat = x.reshape(-1)
    tab_flat = jnp.pad(table, ((0, _VPAD - table.shape[0]), (0, 0))).reshape(-1)
    out = _sc_embed(x_flat, tab_flat)
    return out.reshape(16, 8, _E, 128, 128)


# speedup vs baseline: 26.7359x; 2.2731x over previous
import functools

import jax
import jax.numpy as jnp
from jax import lax
from jax.experimental import pallas as pl
from jax.experimental.pallas import tpu as pltpu
from jax.experimental.pallas import tpu_sc as plsc

_info = plsc.get_sparse_core_info()
_NC, _NS, _L = _info.num_cores, _info.num_subcores, _info.num_lanes
_NW = _NC * _NS

_BT = 128
_E = 16
_HW = 128 * 128
_VPAD = 1008
_CH = 2048
_BT_PER_W = _BT // _NW
_NCH = _HW // _CH
_CHUNKS = _BT_PER_W * _NCH


def _sc_embed(x_flat, tab_flat):
    mesh = plsc.VectorSubcoreMesh(core_axis_name="c", subcore_axis_name="s")

    @functools.partial(
        pl.kernel,
        out_type=jax.ShapeDtypeStruct((_BT, _E, _HW), jnp.float32),
        mesh=mesh,
        compiler_params=pltpu.CompilerParams(needs_layout_passes=False),
        scratch_types=[
            pltpu.VMEM((_VPAD * _E,), jnp.float32),
            pltpu.VMEM((2, _CH), jnp.int32),
            pltpu.VMEM((2, _E, _CH), jnp.float32),
            pltpu.SemaphoreType.DMA,
            pltpu.SemaphoreType.DMA,
            pltpu.SemaphoreType.DMA,
            pltpu.SemaphoreType.DMA,
        ],
    )
    def body(x_hbm, tab_hbm, out_hbm, tab_v, idx_v, out_v,
             isem0, isem1, osem0, osem1):
        wid = lax.axis_index("s") * _NC + lax.axis_index("c")
        isems = (isem0, isem1)
        osems = (osem0, osem1)

        pltpu.sync_copy(tab_hbm, tab_v)

        def idx_off(t):
            bt = wid * _BT_PER_W + (t // _NCH)
            return bt * _HW + (t % _NCH) * _CH

        def start_idx(t, b):
            return pltpu.async_copy(
                x_hbm.at[pl.ds(idx_off(t), _CH)], idx_v.at[b], isems[b])

        def wait_idx(b):
            pltpu.make_async_copy(
                x_hbm.at[pl.ds(0, _CH)], idx_v.at[b], isems[b]).wait()

        def start_out(t, b):
            bt = wid * _BT_PER_W + (t // _NCH)
            dst = out_hbm.at[bt, :, pl.ds((t % _NCH) * _CH, _CH)]
            return pltpu.async_copy(out_v.at[b], dst, osems[b])

        def drain_out(b):
            pltpu.make_async_copy(
                out_v.at[b], out_hbm.at[0, :, pl.ds(0, _CH)], osems[b]).wait()

        def compute(b):
            @plsc.parallel_loop(0, _CH // _L, 1, unroll=2)
            def _(j):
                pos = j * _L
                idx = idx_v[b, pl.ds(pos, _L)]
                base = idx * _E
                vals = [plsc.load_gather(tab_v, [base + e])
                        for e in range(_E)]
                for e in range(_E):
                    out_v[b, e, pl.ds(pos, _L)] = vals[e]

        start_idx(0, 0)
        start_idx(1, 1)

        def outer(tt, carry):
            for b in (0, 1):
                t = tt * 2 + b
                wait_idx(b)

                @pl.when(tt > 0)
                def _():
                    drain_out(b)

                compute(b)
                start_out(t, b)

                @pl.when(t + 2 < _CHUNKS)
                def _():
                    start_idx(t + 2, b)
            return carry

        lax.fori_loop(0, _CHUNKS // 2, outer, 0)
        drain_out(0)
        drain_out(1)

    return body(x_flat, tab_flat)


def kernel(x, table):
    x_flat = x.reshape(-1)
    tab_flat = jnp.pad(table, ((0, _VPAD - table.shape[0]), (0, 0))).reshape(-1)
    out = _sc_embed(x_flat, tab_flat)
    return out.reshape(16, 8, _E, 128, 128)


# speedup vs baseline: 39.3816x; 1.4730x over previous
import functools

import jax
import jax.numpy as jnp
from jax import lax
from jax.experimental import pallas as pl
from jax.experimental.pallas import tpu as pltpu
from jax.experimental.pallas import tpu_sc as plsc

_info = plsc.get_sparse_core_info()
_NC, _NS, _L = _info.num_cores, _info.num_subcores, _info.num_lanes
_NW = _NC * _NS

_BT = 128
_E = 16
_HW = 128 * 128
_VPAD = 1008
_CH = 2048
_BT_PER_W = _BT // _NW
_NCH = _HW // _CH
_CHUNKS = _BT_PER_W * _NCH


def _sc_embed(x_flat, tab_flat):
    mesh = plsc.VectorSubcoreMesh(core_axis_name="c", subcore_axis_name="s")

    @functools.partial(
        pl.kernel,
        out_type=jax.ShapeDtypeStruct((16, 8, _E, 128, 128), jnp.float32),
        mesh=mesh,
        compiler_params=pltpu.CompilerParams(needs_layout_passes=False),
        scratch_types=[
            pltpu.VMEM((_VPAD * _E,), jnp.float32),
            pltpu.VMEM((2, _CH), jnp.int32),
            pltpu.VMEM((2, _E, _CH // 128, 128), jnp.float32),
            pltpu.SemaphoreType.DMA,
            pltpu.SemaphoreType.DMA,
            pltpu.SemaphoreType.DMA,
            pltpu.SemaphoreType.DMA,
        ],
    )
    def body(x_hbm, tab_hbm, out_hbm, tab_v, idx_v, out_v,
             isem0, isem1, osem0, osem1):
        wid = lax.axis_index("s") * _NC + lax.axis_index("c")
        isems = (isem0, isem1)
        osems = (osem0, osem1)

        pltpu.sync_copy(tab_hbm, tab_v)

        def idx_off(t):
            bt = wid * _BT_PER_W + (t // _NCH)
            return bt * _HW + (t % _NCH) * _CH

        def start_idx(t, b):
            return pltpu.async_copy(
                x_hbm.at[pl.ds(idx_off(t), _CH)], idx_v.at[b], isems[b])

        def wait_idx(b):
            pltpu.make_async_copy(
                x_hbm.at[pl.ds(0, _CH)], idx_v.at[b], isems[b]).wait()

        def start_out(t, b):
            bt = wid * _BT_PER_W + (t // _NCH)
            h0 = (t % _NCH) * (_CH // 128)
            dst = out_hbm.at[bt // 8, bt % 8, :, pl.ds(h0, _CH // 128), :]
            return pltpu.async_copy(out_v.at[b], dst, osems[b])

        def drain_out(b):
            pltpu.make_async_copy(
                out_v.at[b],
                out_hbm.at[0, 0, :, pl.ds(0, _CH // 128), :],
                osems[b]).wait()

        def compute(b):
            @plsc.parallel_loop(0, _CH // _L, 1, unroll=2)
            def _(j):
                pos = j * _L
                idx = idx_v[b, pl.ds(pos, _L)]
                base = idx * _E
                vals = [plsc.load_gather(tab_v, [base + e])
                        for e in range(_E)]
                h = j // 8
                w0 = (j % 8) * _L
                for e in range(_E):
                    out_v[b, e, h, pl.ds(w0, _L)] = vals[e]

        start_idx(0, 0)
        start_idx(1, 1)

        def outer(tt, carry):
            for b in (0, 1):
                t = tt * 2 + b
                wait_idx(b)

                @pl.when(tt > 0)
                def _():
                    drain_out(b)

                compute(b)
                start_out(t, b)

                @pl.when(t + 2 < _CHUNKS)
                def _():
                    start_idx(t + 2, b)
            return carry

        lax.fori_loop(0, _CHUNKS // 2, outer, 0)
        drain_out(0)
        drain_out(1)

    return body(x_flat, tab_flat)


def kernel(x, table):
    x_flat = x.reshape(-1)
    tab_flat = jnp.pad(table, ((0, _VPAD - table.shape[0]), (0, 0))).reshape(-1)
    return _sc_embed(x_flat, tab_flat)


# speedup vs baseline: 88.4730x; 2.2466x over previous
import functools

import jax
import jax.numpy as jnp
from jax import lax
from jax.experimental import pallas as pl
from jax.experimental.pallas import tpu as pltpu
from jax.experimental.pallas import tpu_sc as plsc

_info = plsc.get_sparse_core_info()
_NC, _NS, _L = _info.num_cores, _info.num_subcores, _info.num_lanes
_NW = _NC * _NS

_BT = 128
_E = 16
_HW = 128 * 128
_VPAD = 1008
_CH = 2048
_BT_PER_W = _BT // _NW
_NCH = _HW // _CH
_CHUNKS = _BT_PER_W * _NCH


def _sc_embed(x_flat, tab_flat):
    mesh = plsc.VectorSubcoreMesh(core_axis_name="c", subcore_axis_name="s")

    @functools.partial(
        pl.kernel,
        out_type=jax.ShapeDtypeStruct((16, 8, _E, 128, 128), jnp.float32),
        mesh=mesh,
        compiler_params=pltpu.CompilerParams(needs_layout_passes=False),
        scratch_types=[
            pltpu.VMEM((_VPAD * _E,), jnp.float32),
            pltpu.VMEM((2, _CH), jnp.int32),
            pltpu.VMEM((2, _E, _CH // 128, 128), jnp.float32),
            pltpu.SemaphoreType.DMA,
            pltpu.SemaphoreType.DMA,
            pltpu.SemaphoreType.DMA,
            pltpu.SemaphoreType.DMA,
        ],
    )
    def body(x_hbm, tab_hbm, out_hbm, tab_v, idx_v, out_v,
             isem0, isem1, osem0, osem1):
        wid = lax.axis_index("s") * _NC + lax.axis_index("c")
        isems = (isem0, isem1)
        osems = (osem0, osem1)

        pltpu.sync_copy(tab_hbm, tab_v)

        def idx_off(t):
            bt = wid * _BT_PER_W + (t // _NCH)
            return bt * _HW + (t % _NCH) * _CH

        def start_idx(t, b):
            return pltpu.async_copy(
                x_hbm.at[pl.ds(idx_off(t), _CH)], idx_v.at[b], isems[b])

        def wait_idx(b):
            pltpu.make_async_copy(
                x_hbm.at[pl.ds(0, _CH)], idx_v.at[b], isems[b]).wait()

        def start_out(t, b):
            bt = wid * _BT_PER_W + (t // _NCH)
            h0 = (t % _NCH) * (_CH // 128)
            dst = out_hbm.at[bt // 8, bt % 8, :, pl.ds(h0, _CH // 128), :]
            return pltpu.async_copy(out_v.at[b], dst, osems[b])

        def drain_out(b):
            pltpu.make_async_copy(
                out_v.at[b],
                out_hbm.at[0, 0, :, pl.ds(0, _CH // 128), :],
                osems[b]).wait()

        def compute(b):
            @plsc.parallel_loop(0, _CH // _L, 1, unroll=2)
            def _(j):
                pos = j * _L
                idx = idx_v[b, pl.ds(pos, _L)]
                vals = [plsc.load_gather(tab_v, [idx + e * _VPAD])
                        for e in range(_E)]
                h = j // 8
                w0 = (j % 8) * _L
                for e in range(_E):
                    out_v[b, e, h, pl.ds(w0, _L)] = vals[e]

        start_idx(0, 0)
        start_idx(1, 1)

        def outer(tt, carry):
            for b in (0, 1):
                t = tt * 2 + b
                wait_idx(b)

                @pl.when(tt > 0)
                def _():
                    drain_out(b)

                compute(b)
                start_out(t, b)

                @pl.when(t + 2 < _CHUNKS)
                def _():
                    start_idx(t + 2, b)
            return carry

        lax.fori_loop(0, _CHUNKS // 2, outer, 0)
        drain_out(0)
        drain_out(1)

    return body(x_flat, tab_flat)


def kernel(x, table):
    x_flat = x.reshape(-1)
    tab_flat = jnp.pad(table, ((0, _VPAD - table.shape[0]), (0, 0))).T.reshape(-1)
    return _sc_embed(x_flat, tab_flat)
